# baseline (speedup 1.0000x reference)
import functools

import jax
import jax.numpy as jnp
from jax import lax
from jax.experimental import pallas as pl
from jax.experimental.pallas import tpu as pltpu
from jax.experimental.pallas import tpu_sc as plsc

N = 10000
E = 320000
D = 128
H = 128
O = 64

NC = 2
NS = 16
NW = NC * NS
RPT = 320
NPAD = NW * RPT
CH = 1280
NCHUNK = E // CH
G = 64


def _sc_agg_body(x_hbm, edge_hbm, sum_hbm, max_hbm, deg_hbm,
                 src_v, dst_v, sel_src, sel_dst, rows_v,
                 sum_acc, max_acc, deg_acc, sem):
    wid = lax.axis_index("s") * NC + lax.axis_index("c")
    lo = wid * RPT

    zero16 = jnp.zeros((16,), jnp.float32)
    ninf16 = jnp.full((16,), -jnp.inf, jnp.float32)
    izero16 = jnp.zeros((16,), jnp.int32)
    ione16 = jnp.ones((16,), jnp.int32)
    dump16 = jnp.full((16,), RPT, jnp.int32)

    def _init_row(r, _):
        for k in range(D // 16):
            sum_acc[r, pl.ds(k * 16, 16)] = zero16
            max_acc[r, pl.ds(k * 16, 16)] = ninf16
        return 0
    lax.fori_loop(0, RPT + 1, _init_row, 0)

    def _init_small(i, _):
        deg_acc[pl.ds(i * 16, 16)] = izero16
        return 0
    lax.fori_loop(0, RPT // 16, _init_small, 0)

    def _init_sel(i, _):
        sel_src[pl.ds(i * 16, 16)] = izero16
        return 0
    lax.fori_loop(0, (CH + G) // 16, _init_sel, 0)

    def _chunk(c, _):
        base = c * CH
        pltpu.sync_copy(edge_hbm.at[0, pl.ds(base, CH)], src_v)
        pltpu.sync_copy(edge_hbm.at[1, pl.ds(base, CH)], dst_v)

        def _filt(i, n_sel):
            d16 = dst_v[pl.ds(i * 16, 16)]
            s16 = src_v[pl.ds(i * 16, 16)]
            m = (d16 >= lo) & (d16 < lo + RPT)
            dl16 = d16 - lo
            cs = plsc.cumsum(m.astype(jnp.int32))
            pos = n_sel + cs - 1
            plsc.store_scatter(sel_src, [pos], s16, mask=m)
            plsc.store_scatter(sel_dst, [pos], dl16, mask=m)
            plsc.addupdate_scatter(deg_acc, [dl16], ione16, mask=m)
            return n_sel + cs[15]
        n_sel = lax.fori_loop(0, CH // 16, _filt, 0)

        for t in range(G // 16):
            sel_dst[pl.ds(n_sel + t * 16, 16)] = dump16

        nblocks = (n_sel + G - 1) // G

        def _block(b, _):
            pltpu.async_copy(x_hbm.at[sel_src.at[pl.ds(b * G, G)]],
                             rows_v, sem).wait()

            def _sub(j, _):
                dv = sel_dst[pl.ds(b * G + j * 16, 16)]
                for i in range(16):
                    s = dv[i]
                    for k in range(D // 16):
                        r = rows_v[j * 16 + i, pl.ds(k * 16, 16)]
                        a = sum_acc[s, pl.ds(k * 16, 16)]
                        sum_acc[s, pl.ds(k * 16, 16)] = a + r
                        mx = max_acc[s, pl.ds(k * 16, 16)]
                        max_acc[s, pl.ds(k * 16, 16)] = jnp.maximum(mx, r)
                return 0
            lax.fori_loop(0, G // 16, _sub, 0)
            return 0
        lax.fori_loop(0, nblocks, _block, 0)
        return 0
    lax.fori_loop(0, NCHUNK, _chunk, 0)

    pltpu.sync_copy(sum_acc.at[pl.ds(0, RPT)], sum_hbm.at[pl.ds(lo, RPT)])
    pltpu.sync_copy(max_acc.at[pl.ds(0, RPT)], max_hbm.at[pl.ds(lo, RPT)])
    pltpu.sync_copy(deg_acc, deg_hbm.at[wid])


@jax.jit
def _sc_aggregate(x, edge_index):
    mesh = plsc.VectorSubcoreMesh(core_axis_name="c", subcore_axis_name="s")
    k = functools.partial(
        pl.kernel, mesh=mesh,
        compiler_params=pltpu.CompilerParams(needs_layout_passes=False),
        out_type=(
            jax.ShapeDtypeStruct((NPAD, D), jnp.float32),
            jax.ShapeDtypeStruct((NPAD, D), jnp.float32),
            jax.ShapeDtypeStruct((NW, RPT), jnp.int32),
        ),
        scratch_types=[
            pltpu.VMEM((CH,), jnp.int32),
            pltpu.VMEM((CH,), jnp.int32),
            pltpu.VMEM((CH + G,), jnp.int32),
            pltpu.VMEM((CH + G,), jnp.int32),
            pltpu.VMEM((G, D), jnp.float32),
            pltpu.VMEM((RPT + 1, D), jnp.float32),
            pltpu.VMEM((RPT + 1, D), jnp.float32),
            pltpu.VMEM((RPT,), jnp.int32),
            pltpu.SemaphoreType.DMA,
        ],
    )(_sc_agg_body)
    return k(x, edge_index)


def _tc_dense_body(sum_ref, max_ref, deg_ref, x_ref,
                   wlm_ref, wrm_ref, wlx_ref, wrx_ref, wpl_ref, wpr_ref,
                   bm_ref, bx_ref, bp_ref, out_ref):
    deg = jnp.maximum(deg_ref[...].astype(jnp.float32), 1.0)
    mean = sum_ref[...] / deg
    mx = max_ref[...]
    mx = jnp.where(jnp.isneginf(mx), 0.0, mx)
    xb = x_ref[...]
    hm = jnp.dot(mean, wlm_ref[...], preferred_element_type=jnp.float32)
    hm = hm + jnp.dot(xb, wrm_ref[...], preferred_element_type=jnp.float32)
    hm = hm + bm_ref[...]
    hx = jnp.dot(mx, wlx_ref[...], preferred_element_type=jnp.float32)
    hx = hx + jnp.dot(xb, wrx_ref[...], preferred_element_type=jnp.float32)
    hx = hx + bx_ref[...]
    logits = jnp.dot(hm, wpl_ref[...], preferred_element_type=jnp.float32)
    logits = logits + jnp.dot(hx, wpr_ref[...], preferred_element_type=jnp.float32)
    logits = logits + bp_ref[...]
    m = jnp.max(logits, axis=-1, keepdims=True)
    z = logits - m
    out_ref[...] = z - jnp.log(jnp.sum(jnp.exp(z), axis=-1, keepdims=True))


@jax.jit
def _tc_dense(sum_agg, max_agg, deg, xp, wlm, wrm, wlx, wrx, wpl, wpr,
              bm, bx, bp):
    BR = 256
    grid = (NPAD // BR,)
    blk = lambda i: (i, 0)
    fix = lambda i: (0, 0)
    return pl.pallas_call(
        _tc_dense_body,
        grid=grid,
        in_specs=[
            pl.BlockSpec((BR, D), blk),
            pl.BlockSpec((BR, D), blk),
            pl.BlockSpec((BR, 1), blk),
            pl.BlockSpec((BR, D), blk),
            pl.BlockSpec((D, H), fix),
            pl.BlockSpec((D, H), fix),
            pl.BlockSpec((D, H), fix),
            pl.BlockSpec((D, H), fix),
            pl.BlockSpec((H, O), fix),
            pl.BlockSpec((H, O), fix),
            pl.BlockSpec((1, H), fix),
            pl.BlockSpec((1, H), fix),
            pl.BlockSpec((1, O), fix),
        ],
        out_specs=pl.BlockSpec((BR, O), blk),
        out_shape=jax.ShapeDtypeStruct((NPAD, O), jnp.float32),
    )(sum_agg, max_agg, deg, xp, wlm, wrm, wlx, wrx, wpl, wpr, bm, bx, bp)


def kernel(x, edge_index, Wl_mean, Wr_mean, b_mean, Wl_max, Wr_max, b_max,
           W_post, b_post):
    sum_agg, max_agg, deg2d = _sc_aggregate(x, edge_index)
    deg = deg2d.reshape(NPAD, 1)
    xp = jnp.pad(x, ((0, NPAD - N), (0, 0)))
    out = _tc_dense(sum_agg, max_agg, deg, xp,
                    Wl_mean.T, Wr_mean.T, Wl_max.T, Wr_max.T,
                    W_post[:, :H].T, W_post[:, H:].T,
                    b_mean.reshape(1, H), b_max.reshape(1, H),
                    b_post.reshape(1, O))
    return out[:N]
